# baseline (speedup 1.0000x reference)
import functools

import jax
import jax.numpy as jnp
from jax import lax
from jax.experimental import pallas as pl
from jax.experimental.pallas import tpu as pltpu
from jax.experimental.pallas import tpu_sc as plsc

NUM_ROWS = 4096 * 50
DIM = 32
NC = 2
NS = 16
NW = NC * NS
B_PER_W = NUM_ROWS // NW
IW = 128
G = B_PER_W // IW
K = 10
NG = G // K


@functools.partial(
    pl.kernel,
    mesh=plsc.VectorSubcoreMesh(
        core_axis_name="c", subcore_axis_name="s", num_cores=NC, num_subcores=NS
    ),
    out_type=jax.ShapeDtypeStruct((NUM_ROWS // IW, IW, DIM), jnp.float32),
    compiler_params=pltpu.CompilerParams(use_tc_tiling_on_sc=False),
    scratch_types=[
        pltpu.VMEM((G, IW), jnp.int32),
        pltpu.VMEM((2, K, IW, DIM), jnp.float32),
        pltpu.SemaphoreType.DMA,
        pltpu.SemaphoreType.DMA,
    ],
)
def _gather_kernel(idx_hbm, table_hbm, out_hbm, idx_v, rows_v, sem0, sem1):
    wid = lax.axis_index("s") * NC + lax.axis_index("c")
    sems = (sem0, sem1)
    pltpu.sync_copy(idx_hbm.at[wid], idx_v)

    def fire(g, buf):
        for j in range(K):
            pltpu.async_copy(
                table_hbm.at[idx_v.at[g * K + j]], rows_v.at[buf, j], sems[buf]
            )

    fire(0, 0)
    for g in range(NG):
        buf = g % 2
        if g + 1 < NG:
            fire(g + 1, (g + 1) % 2)
        for j in range(K):
            pltpu.make_async_copy(
                table_hbm.at[idx_v.at[g * K + j]], rows_v.at[buf, j], sems[buf]
            ).wait()
        pltpu.sync_copy(
            rows_v.at[buf], out_hbm.at[pl.ds(wid * G + g * K, K)]
        )


def kernel(model_input, weight):
    idx = model_input.reshape(NW, G, IW).astype(jnp.int32)
    out = _gather_kernel(idx, weight)
    return out.reshape(model_input.shape + (DIM,))


# speedup vs baseline: 1.0350x; 1.0350x over previous
import functools

import jax
import jax.numpy as jnp
from jax import lax
from jax.experimental import pallas as pl
from jax.experimental.pallas import tpu as pltpu
from jax.experimental.pallas import tpu_sc as plsc

BATCH = 4096
SEQ = 50
DIM = 32
NC = 2
NS = 16
NW = NC * NS
BW = BATCH // NW


@functools.partial(
    pl.kernel,
    mesh=plsc.VectorSubcoreMesh(
        core_axis_name="c", subcore_axis_name="s", num_cores=NC, num_subcores=NS
    ),
    out_type=jax.ShapeDtypeStruct((SEQ, DIM, BATCH), jnp.float32),
    compiler_params=pltpu.CompilerParams(
        use_tc_tiling_on_sc=False, needs_layout_passes=False
    ),
    scratch_types=[
        pltpu.VMEM((SEQ, BW), jnp.int32),
        pltpu.VMEM((2, BW, DIM), jnp.float32),
        pltpu.VMEM((DIM, BW), jnp.float32),
        pltpu.SemaphoreType.DMA,
        pltpu.SemaphoreType.DMA,
    ],
)
def _gather_kernel(idx_hbm, table_hbm, out_hbm, idx_v, gbuf, tbuf, sem0, sem1):
    wid = lax.axis_index("s") * NC + lax.axis_index("c")
    col0 = wid * BW
    sems = (sem0, sem1)
    pltpu.sync_copy(idx_hbm.at[:, pl.ds(col0, BW)], idx_v)

    lanes = lax.iota(jnp.int32, 16)

    def fire(s, buf):
        pltpu.async_copy(table_hbm.at[idx_v.at[s]], gbuf.at[buf], sems[buf])

    def drain(s, buf):
        pltpu.make_async_copy(
            table_hbm.at[idx_v.at[s]], gbuf.at[buf], sems[buf]
        ).wait()

    def emit(s, buf):
        for d in range(DIM):
            cols = jnp.full((16,), d, jnp.int32)
            for g in range(BW // 16):
                rows = lanes + (g * 16)
                vals = plsc.load_gather(gbuf.at[buf], [rows, cols])
                tbuf[d, pl.ds(g * 16, 16)] = vals
        pltpu.sync_copy(tbuf, out_hbm.at[s].at[:, pl.ds(col0, BW)])

    fire(0, 0)

    def body(p, carry):
        s = p * 2
        fire(s + 1, 1)
        drain(s, 0)
        emit(s, 0)

        @pl.when(s + 2 < SEQ)
        def _():
            fire(s + 2, 0)

        drain(s + 1, 1)
        emit(s + 1, 1)
        return carry

    lax.fori_loop(0, SEQ // 2, body, 0)


def kernel(model_input, weight):
    idx_t = model_input.T.astype(jnp.int32)
    out_t = _gather_kernel(idx_t, weight)
    return out_t.transpose(2, 0, 1)


# speedup vs baseline: 1.5679x; 1.5149x over previous
import functools

import jax
import jax.numpy as jnp
from jax import lax
from jax.experimental import pallas as pl
from jax.experimental.pallas import tpu as pltpu
from jax.experimental.pallas import tpu_sc as plsc

BATCH = 4096
SEQ = 50
DIM = 32
VOCAB = 1000000
NC = 2
NS = 16
NW = NC * NS
BW = BATCH // NW

NFULL = VOCAB // 128
VPAD = (NFULL + 1) * 128
LROWS = VPAD * DIM // 128
TAIL0 = NFULL * 128
LTAIL0 = TAIL0 * DIM // 128

_MESH = plsc.VectorSubcoreMesh(
    core_axis_name="c", subcore_axis_name="s", num_cores=NC, num_subcores=NS
)
_LANES = None


@functools.partial(
    pl.kernel,
    mesh=_MESH,
    out_type=jax.ShapeDtypeStruct((LROWS, 128), jnp.float32),
    compiler_params=pltpu.CompilerParams(needs_layout_passes=False),
    scratch_types=[
        pltpu.VMEM((2, DIM, 128), jnp.float32),
        pltpu.VMEM((2, DIM, 128), jnp.float32),
        pltpu.VMEM((16, 128), jnp.float32),
        pltpu.SemaphoreType.DMA,
        pltpu.SemaphoreType.DMA,
        pltpu.SemaphoreType.DMA,
        pltpu.SemaphoreType.DMA,
    ],
)
def _relayout_kernel(wt_hbm, tail_hbm, lin_hbm, gtile, ttile, tailv, r0, r1, w0, w1):
    wid = lax.axis_index("s") * NC + lax.axis_index("c")
    rsems = (r0, r1)
    wsems = (w0, w1)
    lanes = lax.iota(jnp.int32, 16)
    lanes32 = lanes * 32

    def blk(t):
        return t * NW + wid

    def rd_desc(t, buf):
        cb = pl.multiple_of(blk(t) * 128, 128)
        return pltpu.make_async_copy(
            wt_hbm.at[:, pl.ds(cb, 128)], gtile.at[buf], rsems[buf]
        )

    def wr_desc(t, buf):
        r = pl.multiple_of(blk(t) * DIM, DIM)
        return pltpu.make_async_copy(
            ttile.at[buf], lin_hbm.at[pl.ds(r, DIM)], wsems[buf]
        )

    def transpose(buf):
        src = gtile.at[buf]
        dst = ttile.at[buf]
        for d0 in range(DIM):
            rows_src = jnp.bitwise_and(lanes + d0, DIM - 1)
            base = lanes32 + rows_src
            for g in range(8):
                cols_src = lanes + (g * 16)
                vals = plsc.load_gather(src, [rows_src, cols_src])
                flat = base + (g * 512)
                plsc.store_scatter(
                    dst,
                    [lax.shift_right_logical(flat, 7), jnp.bitwise_and(flat, 127)],
                    vals,
                )

    @pl.when(blk(0) < NFULL)
    def _():
        rd_desc(0, 0).start()

    def half(t, buf, nbuf):
        @pl.when(blk(t + 1) < NFULL)
        def _():
            rd_desc(t + 1, nbuf).start()

        @pl.when(blk(t) < NFULL)
        def _():
            rd_desc(t, buf).wait()

            @pl.when(t >= 2)
            def _():
                wr_desc(t - 2, buf).wait()

            transpose(buf)
            wr_desc(t, buf).start()

    def body(p, carry):
        t = p * 2
        half(t, 0, 1)
        half(t + 1, 1, 0)
        return carry

    lax.fori_loop(0, 123, body, 0)

    for tt in (242, 243, 244):
        @pl.when((blk(tt) < NFULL) & (blk(tt + 2) >= NFULL))
        def _(tt=tt):
            wr_desc(tt, tt % 2).wait()

    @pl.when(wid == 0)
    def _():
        pltpu.sync_copy(tail_hbm, tailv)
        pltpu.sync_copy(tailv, lin_hbm.at[pl.ds(LTAIL0, 16)])


@functools.partial(
    pl.kernel,
    mesh=_MESH,
    out_type=jax.ShapeDtypeStruct((SEQ, DIM, BATCH), jnp.float32),
    compiler_params=pltpu.CompilerParams(
        use_tc_tiling_on_sc=False, needs_layout_passes=False
    ),
    scratch_types=[
        pltpu.VMEM((SEQ, BW), jnp.int32),
        pltpu.VMEM((2, BW, DIM), jnp.float32),
        pltpu.VMEM((2, DIM, BW), jnp.float32),
        pltpu.SemaphoreType.DMA,
        pltpu.SemaphoreType.DMA,
        pltpu.SemaphoreType.DMA,
        pltpu.SemaphoreType.DMA,
    ],
)
def _gather_kernel(idx_hbm, table_hbm, out_hbm, idx_v, gbuf, tbuf, g0, g1, w0, w1):
    wid = lax.axis_index("s") * NC + lax.axis_index("c")
    col0 = wid * BW
    gsems = (g0, g1)
    wsems = (w0, w1)
    lanes = lax.iota(jnp.int32, 16)

    pltpu.sync_copy(idx_hbm.at[:, pl.ds(col0, BW)], idx_v)

    def g_desc(s, buf):
        return pltpu.make_async_copy(
            table_hbm.at[idx_v.at[s]], gbuf.at[buf], gsems[buf]
        )

    def w_desc(s, buf):
        return pltpu.make_async_copy(
            tbuf.at[buf], out_hbm.at[s].at[:, pl.ds(col0, BW)], wsems[buf]
        )

    def transpose(buf):
        src = gbuf.at[buf]
        dst = tbuf.at[buf]
        for d0 in range(DIM):
            rows2 = jnp.bitwise_and(lanes + d0, DIM - 1)
            for g in range(BW // 16):
                cols2 = lanes + (g * 16)
                vals = plsc.load_gather(src, [cols2, rows2])
                plsc.store_scatter(dst, [rows2, cols2], vals)

    g_desc(0, 0).start()

    def half(s, buf, nbuf):
        @pl.when(s + 1 < SEQ)
        def _():
            g_desc(s + 1, nbuf).start()

        g_desc(s, buf).wait()

        @pl.when(s >= 2)
        def _():
            w_desc(s - 2, buf).wait()

        transpose(buf)
        w_desc(s, buf).start()

    def body(p, carry):
        s = p * 2
        half(s, 0, 1)
        half(s + 1, 1, 0)
        return carry

    lax.fori_loop(0, SEQ // 2, body, 0)
    w_desc(SEQ - 2, 0).wait()
    w_desc(SEQ - 1, 1).wait()


def kernel(model_input, weight):
    w_t = weight.T
    w_tail = lax.slice(weight, (TAIL0, 0), (VOCAB, DIM)).reshape(16, 128)
    w_lin = _relayout_kernel(w_t, w_tail)
    table = w_lin.reshape(VPAD, DIM)
    idx_t = model_input.T.astype(jnp.int32)
    out_t = _gather_kernel(idx_t, table)
    return out_t.transpose(2, 0, 1)
